# baseline (speedup 1.0000x reference)
import functools

import jax
import jax.numpy as jnp
from jax import lax
from jax.experimental import pallas as pl
from jax.experimental.pallas import tpu as pltpu
from jax.experimental.pallas import tpu_sc as plsc

N_NODES = 10000
D = 128
N_EDGES = 320000

NC = 2
NS = 16
CHUNK = 80
RING = 4
N_ITERS = 125

EDGES_PAD = NC * NS * N_ITERS * CHUNK

ROWS_PER_TILE = 624


def _make_sc_kernel():
    mesh = plsc.VectorSubcoreMesh(core_axis_name="c", subcore_axis_name="s")

    @functools.partial(
        pl.kernel,
        out_type=jax.ShapeDtypeStruct((NC * N_NODES, D), jnp.float32),
        mesh=mesh,
        scratch_types=[
            pltpu.VMEM_SHARED((N_NODES, D), jnp.float32),
            [pltpu.VMEM((2, CHUNK), jnp.int32) for _ in range(RING)],
            [pltpu.VMEM((CHUNK, D), jnp.float32) for _ in range(RING)],
            [pltpu.SemaphoreType.DMA for _ in range(RING)],
            [pltpu.SemaphoreType.DMA for _ in range(RING)],
            [pltpu.SemaphoreType.DMA for _ in range(RING)],
            pltpu.SemaphoreType.DMA,
        ],
    )
    def sc_kernel(x_hbm, eidx_hbm, out_hbm, accum, ebuf, rows,
                  semg, semg2, sems, semz):
        c = lax.axis_index("c")
        s = lax.axis_index("s")
        w = c * NS + s
        row0 = s * ROWS_PER_TILE

        def idxcopy(i, b):
            pltpu.sync_copy(eidx_hbm.at[w, i], ebuf[b])

        H = CHUNK // 2

        def gather(b):
            pltpu.async_copy(x_hbm.at[ebuf[b].at[0, pl.ds(0, H)]],
                             rows[b].at[pl.ds(0, H)], semg[b])
            pltpu.async_copy(x_hbm.at[ebuf[b].at[0, pl.ds(H, H)]],
                             rows[b].at[pl.ds(H, H)], semg2[b])

        def scatter(b):
            return pltpu.async_copy(rows[b], accum.at[ebuf[b].at[1]],
                                    sems[b], add=True)

        def wait_gather(b):
            pltpu.make_async_copy(x_hbm.at[ebuf[b].at[0, pl.ds(0, H)]],
                                  rows[b].at[pl.ds(0, H)], semg[b]).wait()
            pltpu.make_async_copy(x_hbm.at[ebuf[b].at[0, pl.ds(H, H)]],
                                  rows[b].at[pl.ds(H, H)], semg2[b]).wait()

        def wait_scatter(b):
            pltpu.make_async_copy(rows[b], accum.at[ebuf[b].at[1]],
                                  sems[b]).wait()

        P = RING - 1

        for j in range(P):
            idxcopy(j, j)
            gather(j)

        zsrc = rows[P].at[pl.ds(0, 16)]
        zero_v = jnp.zeros((16,), jnp.float32)
        for r in range(16):
            for j in range(D // 16):
                rows[P][r, pl.ds(j * 16, 16)] = zero_v

        nz = jnp.where(s == NS - 1, 40, 39)

        @pl.loop(0, nz)
        def _zero(i):
            pltpu.async_copy(zsrc, accum.at[pl.ds(row0 + i * 16, 16)], semz)

        @pl.loop(0, nz)
        def _zwait(i):
            pltpu.make_async_copy(zsrc, accum.at[pl.ds(row0, 16)],
                                  semz).wait()

        plsc.subcore_barrier()

        @pl.loop(0, N_ITERS // RING)
        def _steps(g):
            for b in range(RING):
                i = RING * g + b
                pb = (b + P) % RING
                wait_gather(b)
                scatter(b)

                @pl.when(i + P < N_ITERS)
                def _prefetch():
                    if b == 0:
                        @pl.when(i >= 1)
                        def _w():
                            wait_scatter(pb)
                    else:
                        wait_scatter(pb)
                    idxcopy(i + P, pb)
                    gather(pb)

        n_full = (N_ITERS // RING) * RING
        for i in range(n_full, N_ITERS):
            wait_gather(i % RING)
            scatter(i % RING)

        for b in range(RING):
            wait_scatter(b)

        plsc.subcore_barrier()

        out_row0 = c * N_NODES + row0
        pltpu.sync_copy(accum.at[pl.ds(row0, ROWS_PER_TILE)],
                        out_hbm.at[pl.ds(out_row0, ROWS_PER_TILE)])

        @pl.when(s == NS - 1)
        def _tail():
            pltpu.sync_copy(accum.at[pl.ds(NS * ROWS_PER_TILE, 16)],
                            out_hbm.at[pl.ds(c * N_NODES + NS * ROWS_PER_TILE,
                                             16)])

    return sc_kernel


_sc_kernel = _make_sc_kernel()


def _combine_body(a_ref, b_ref, o_ref):
    o_ref[...] = a_ref[...] + b_ref[...]


@jax.jit
def _combine(partials):
    blk = 1000
    grid = N_NODES // blk
    return pl.pallas_call(
        _combine_body,
        grid=(grid,),
        in_specs=[
            pl.BlockSpec((blk, D), lambda i: (i, 0)),
            pl.BlockSpec((blk, D), lambda i: (i + N_NODES // blk, 0)),
        ],
        out_specs=pl.BlockSpec((blk, D), lambda i: (i, 0)),
        out_shape=jax.ShapeDtypeStruct((N_NODES, D), jnp.float32),
    )(partials, partials)


@jax.jit
def kernel(x, edge_index):
    pad = EDGES_PAD - N_EDGES
    if pad:
        xp = jnp.concatenate([x, jnp.zeros((8, D), jnp.float32)])
        pad_edges = jnp.stack([
            N_NODES + jnp.arange(pad, dtype=jnp.int32) % 8,
            jnp.arange(pad, dtype=jnp.int32) % N_NODES,
        ])
        eidx = jnp.concatenate([edge_index, pad_edges], axis=1)
    else:
        xp = x
        eidx = edge_index
    eidx = eidx.reshape(2, NC * NS, N_ITERS, CHUNK).transpose(1, 2, 0, 3)
    partials = _sc_kernel(xp, eidx)
    return _combine(partials)


# speedup vs baseline: 1.0597x; 1.0597x over previous
import functools

import jax
import jax.numpy as jnp
from jax import lax
from jax.experimental import pallas as pl
from jax.experimental.pallas import tpu as pltpu
from jax.experimental.pallas import tpu_sc as plsc

N_NODES = 10000
D = 128
N_EDGES = 320000

NC = 2
NS = 16
CHUNK = 80
RING = 4
N_ITERS = 125

EDGES_PAD = NC * NS * N_ITERS * CHUNK

ROWS_PER_TILE = 624


def _make_sc_kernel():
    mesh = plsc.VectorSubcoreMesh(core_axis_name="c", subcore_axis_name="s")

    @functools.partial(
        pl.kernel,
        out_type=jax.ShapeDtypeStruct((NC * N_NODES, D), jnp.float32),
        mesh=mesh,
        scratch_types=[
            pltpu.VMEM_SHARED((N_NODES, D), jnp.float32),
            [pltpu.VMEM((2, CHUNK), jnp.int32) for _ in range(RING)],
            [pltpu.VMEM((CHUNK, D), jnp.float32) for _ in range(RING)],
            [pltpu.SemaphoreType.DMA for _ in range(RING)],
            [pltpu.SemaphoreType.DMA for _ in range(RING)],
            pltpu.SemaphoreType.DMA,
        ],
    )
    def sc_kernel(x_hbm, eidx_hbm, out_hbm, accum, ebuf, rows,
                  semg, sems, semz):
        c = lax.axis_index("c")
        s = lax.axis_index("s")
        w = c * NS + s
        row0 = s * ROWS_PER_TILE

        def idxcopy(i, b):
            pltpu.sync_copy(eidx_hbm.at[w, i], ebuf[b])

        def gather(b):
            return pltpu.async_copy(x_hbm.at[ebuf[b].at[0]], rows[b],
                                    semg[b])

        def scatter(b):
            return pltpu.async_copy(rows[b], accum.at[ebuf[b].at[1]],
                                    sems[b], add=True)

        def wait_gather(b):
            pltpu.make_async_copy(x_hbm.at[ebuf[b].at[0]], rows[b],
                                  semg[b]).wait()

        def wait_scatter(b):
            pltpu.make_async_copy(rows[b], accum.at[ebuf[b].at[1]],
                                  sems[b]).wait()

        P = RING - 1

        for j in range(P):
            idxcopy(j, j)
            gather(j)

        zsrc = rows[P].at[pl.ds(0, 16)]
        zero_v = jnp.zeros((16,), jnp.float32)
        for r in range(16):
            for j in range(D // 16):
                rows[P][r, pl.ds(j * 16, 16)] = zero_v

        nz = jnp.where(s == NS - 1, 40, 39)

        @pl.loop(0, nz)
        def _zero(i):
            pltpu.async_copy(zsrc, accum.at[pl.ds(row0 + i * 16, 16)], semz)

        @pl.loop(0, nz)
        def _zwait(i):
            pltpu.make_async_copy(zsrc, accum.at[pl.ds(row0, 16)],
                                  semz).wait()

        plsc.subcore_barrier()

        @pl.loop(0, N_ITERS // RING)
        def _steps(g):
            for b in range(RING):
                i = RING * g + b
                pb = (b + P) % RING
                wait_gather(b)
                scatter(b)

                @pl.when(i + P < N_ITERS)
                def _prefetch():
                    if b == 0:
                        @pl.when(i >= 1)
                        def _w():
                            wait_scatter(pb)
                    else:
                        wait_scatter(pb)
                    idxcopy(i + P, pb)
                    gather(pb)

        n_full = (N_ITERS // RING) * RING
        for i in range(n_full, N_ITERS):
            wait_gather(i % RING)
            scatter(i % RING)

        for b in range(RING):
            wait_scatter(b)

        plsc.subcore_barrier()

        out_row0 = c * N_NODES + row0
        pltpu.sync_copy(accum.at[pl.ds(row0, ROWS_PER_TILE)],
                        out_hbm.at[pl.ds(out_row0, ROWS_PER_TILE)])

        @pl.when(s == NS - 1)
        def _tail():
            pltpu.sync_copy(accum.at[pl.ds(NS * ROWS_PER_TILE, 16)],
                            out_hbm.at[pl.ds(c * N_NODES + NS * ROWS_PER_TILE,
                                             16)])

    return sc_kernel


_sc_kernel = _make_sc_kernel()


def _combine_body(a_ref, b_ref, o_ref):
    o_ref[...] = a_ref[...] + b_ref[...]


@jax.jit
def _combine(partials):
    blk = 1000
    grid = N_NODES // blk
    return pl.pallas_call(
        _combine_body,
        grid=(grid,),
        in_specs=[
            pl.BlockSpec((blk, D), lambda i: (i, 0)),
            pl.BlockSpec((blk, D), lambda i: (i + N_NODES // blk, 0)),
        ],
        out_specs=pl.BlockSpec((blk, D), lambda i: (i, 0)),
        out_shape=jax.ShapeDtypeStruct((N_NODES, D), jnp.float32),
    )(partials, partials)


@jax.jit
def kernel(x, edge_index):
    pad = EDGES_PAD - N_EDGES
    if pad:
        xp = jnp.concatenate([x, jnp.zeros((8, D), jnp.float32)])
        pad_edges = jnp.stack([
            N_NODES + jnp.arange(pad, dtype=jnp.int32) % 8,
            jnp.arange(pad, dtype=jnp.int32) % N_NODES,
        ])
        eidx = jnp.concatenate([edge_index, pad_edges], axis=1)
    else:
        xp = x
        eidx = edge_index
    eidx = eidx.reshape(2, NC * NS, N_ITERS, CHUNK).transpose(1, 2, 0, 3)
    partials = _sc_kernel(xp, eidx)
    return _combine(partials)
